# baseline (speedup 1.0000x reference)
import jax
import jax.numpy as jnp
from jax import lax
from jax.experimental import pallas as pl
from jax.experimental.pallas import tpu as pltpu
from jax.experimental.pallas import tpu_sc as plsc

N = 10000
E = 320000
D = 128

NC = 2
NS = 16
NW = NC * NS

N_PAD = 10112
ROWS_PER_TILE = N_PAD // NS
CHUNK = 128
CPT0 = 80
CPT1 = 80
CPT_MAX = max(CPT0, CPT1)
IDX_ROWS = NS * (CPT0 + CPT1)
E_PAD = IDX_ROWS * CHUNK
PAD_ROWS = NS * CPT0 + (NS - 1) * CPT1 + CPT_MAX
OUT_STRIDE = CPT_MAX * CHUNK
DUMMY = N
DST_BLK = 8


def _core_offsets(c, s):
    cpt = jnp.where(c == 0, CPT0, CPT1)
    base_row = jnp.where(c == 0, s * CPT0, NS * CPT0 + s * CPT1)
    return cpt, base_row


def _mm_body(x_ref, w_ref, ws_ref, xw_ref, xws_ref):
    xb = x_ref[...]
    xw_ref[...] = jnp.dot(xb, w_ref[...], preferred_element_type=jnp.float32)
    xws_ref[...] = jnp.dot(xb, ws_ref[...], preferred_element_type=jnp.float32)


def _matmuls(x_pad, W, W_self):
    blk = N_PAD // 8
    return pl.pallas_call(
        _mm_body,
        grid=(8,),
        in_specs=[
            pl.BlockSpec((blk, D), lambda i: (i, 0)),
            pl.BlockSpec((D, D), lambda i: (0, 0)),
            pl.BlockSpec((D, D), lambda i: (0, 0)),
        ],
        out_specs=[
            pl.BlockSpec((blk, D), lambda i: (i, 0)),
            pl.BlockSpec((blk, D), lambda i: (i, 0)),
        ],
        out_shape=[
            jax.ShapeDtypeStruct((N_PAD, D), jnp.float32),
            jax.ShapeDtypeStruct((N_PAD, D), jnp.float32),
        ],
    )(x_pad, W, W_self)


def _scatter_body(xw_hbm, xws_hbm, src_hbm, dst_hbm, parts_hbm,
                  src_idx, dst_idx, rows0, rows1, acc, sem0, sem1, sd0, sd1):
    c = lax.axis_index("c")
    s = lax.axis_index("s")
    cpt, base_row = _core_offsets(c, s)

    pltpu.sync_copy(xws_hbm.at[pl.ds(s * ROWS_PER_TILE, ROWS_PER_TILE)],
                    acc.at[pl.ds(s * ROWS_PER_TILE, ROWS_PER_TILE)])

    pltpu.sync_copy(src_hbm.at[pl.ds(base_row, CPT_MAX)], src_idx)
    plsc.subcore_barrier()

    bufs = (rows0, rows1)
    sems = (sem0, sem1)
    sd = (sd0, sd1)
    ngroups = cpt // DST_BLK

    pltpu.async_copy(xw_hbm.at[src_idx.at[0]], rows0, sem0)
    pltpu.async_copy(dst_hbm.at[pl.ds(base_row, DST_BLK)], dst_idx.at[0], sd0)

    def gpair(p, carry):
        for par in range(2):
            grp = 2 * p + par
            nk = (par + 1) % 2

            @pl.when(grp + 1 < ngroups)
            def _stage_next():
                pltpu.async_copy(
                    dst_hbm.at[pl.ds(base_row + (grp + 1) * DST_BLK, DST_BLK)],
                    dst_idx.at[nk], sd[nk])

            pltpu.make_async_copy(dst_hbm.at[pl.ds(base_row, DST_BLK)],
                                  dst_idx.at[par], sd[par]).wait()
            for i in range(DST_BLK):
                j = grp * DST_BLK + i

                @pl.when(j + 1 < cpt)
                def _fire():
                    pltpu.async_copy(xw_hbm.at[src_idx.at[j + 1]],
                                     bufs[(i + 1) % 2], sems[(i + 1) % 2])

                pltpu.make_async_copy(xw_hbm.at[src_idx.at[j]], bufs[i % 2],
                                      sems[i % 2]).wait()
                pltpu.sync_copy(bufs[i % 2], acc.at[dst_idx.at[par].at[i]],
                                add=True)
        return carry

    lax.fori_loop(0, ngroups // 2, gpair, 0)
    plsc.subcore_barrier()

    pltpu.sync_copy(acc.at[pl.ds(s * ROWS_PER_TILE, ROWS_PER_TILE)],
                    parts_hbm.at[c].at[pl.ds(s * ROWS_PER_TILE, ROWS_PER_TILE)])


def _scatter_phase(xw, xws, src2d, dst2d):
    mesh = plsc.VectorSubcoreMesh(core_axis_name="c", subcore_axis_name="s",
                                  num_cores=NC, num_subcores=NS)
    fn = pl.kernel(
        _scatter_body,
        out_type=jax.ShapeDtypeStruct((NC, N_PAD, D), jnp.float32),
        mesh=mesh,
        scratch_types=[
            pltpu.VMEM((CPT_MAX, CHUNK), jnp.int32),
            pltpu.VMEM((2, DST_BLK, CHUNK), jnp.int32),
            pltpu.VMEM((CHUNK, D), jnp.float32),
            pltpu.VMEM((CHUNK, D), jnp.float32),
            pltpu.VMEM_SHARED((N_PAD, D), jnp.float32),
            pltpu.SemaphoreType.DMA,
            pltpu.SemaphoreType.DMA,
            pltpu.SemaphoreType.DMA,
            pltpu.SemaphoreType.DMA,
        ],
    )
    return fn(xw, xws, src2d, dst2d)


def _comb_body(p_ref, xws_ref, h_ref):
    h = jnp.maximum(p_ref[0] + p_ref[1] - xws_ref[...], 0.0)
    h_ref[...] = h.astype(jnp.bfloat16)


def _combine(parts, xws):
    blk = N_PAD // 8
    return pl.pallas_call(
        _comb_body,
        grid=(8,),
        in_specs=[
            pl.BlockSpec((NC, blk, D), lambda i: (0, i, 0)),
            pl.BlockSpec((blk, D), lambda i: (i, 0)),
        ],
        out_specs=pl.BlockSpec((blk, D), lambda i: (i, 0)),
        out_shape=jax.ShapeDtypeStruct((N_PAD, D), jnp.bfloat16),
    )(parts, xws)


def _dot_chunk_compute(ra, rb, part, out_c, lane):
    fmt = plsc.PackFormat.INTERLEAVED
    for g in range(CHUNK // 16):
        for r in range(16):
            row = g * 16 + r
            acc = jnp.zeros((16,), jnp.float32)
            for t in range(D // 32):
                va = plsc.bitcast(ra[row, pl.ds(16 * t, 16)], jnp.bfloat16)
                vb = plsc.bitcast(rb[row, pl.ds(16 * t, 16)], jnp.bfloat16)
                a0, a1 = plsc.unpack(va, format=fmt)
                b0, b1 = plsc.unpack(vb, format=fmt)
                acc = acc + a0 * b0 + a1 * b1
            part[r, :] = acc
        dots = jnp.zeros((16,), jnp.float32)
        for t in range(16):
            col = jnp.full((16,), t, jnp.int32)
            dots = dots + plsc.load_gather(part, [lane, col])
        out_c[pl.ds(g * 16, 16)] = dots


NBUF = 4


def _dot_body(h_hbm, a_hbm, b_hbm, pred_hbm,
              a_idx, b_idx, ra0, rb0, ra1, rb1, ra2, rb2, ra3, rb3,
              part, oc0, oc1,
              sa0, sb0, sa1, sb1, sa2, sb2, sa3, sb3, so0, so1):
    c = lax.axis_index("c")
    s = lax.axis_index("s")
    wid = c * NS + s
    cpt, base_row = _core_offsets(c, s)

    pltpu.sync_copy(a_hbm.at[pl.ds(base_row, CPT_MAX)], a_idx)
    pltpu.sync_copy(b_hbm.at[pl.ds(base_row, CPT_MAX)], b_idx)

    lane = lax.iota(jnp.int32, 16)
    ra = (ra0, ra1, ra2, ra3)
    rb = (rb0, rb1, rb2, rb3)
    sa = (sa0, sa1, sa2, sa3)
    sb = (sb0, sb1, sb2, sb3)
    oc = (oc0, oc1)
    so = (so0, so1)
    out_base = wid * OUT_STRIDE

    for k in range(NBUF - 1):
        pltpu.async_copy(h_hbm.at[a_idx.at[k]], ra[k], sa[k])
        pltpu.async_copy(h_hbm.at[b_idx.at[k]], rb[k], sb[k])

    def quad(q, carry):
        for i in range(NBUF):
            j = NBUF * q + i

            @pl.when(j + NBUF - 1 < cpt)
            def _fire():
                k = (i + NBUF - 1) % NBUF
                pltpu.async_copy(h_hbm.at[a_idx.at[j + NBUF - 1]], ra[k],
                                 sa[k])
                pltpu.async_copy(h_hbm.at[b_idx.at[j + NBUF - 1]], rb[k],
                                 sb[k])

            pltpu.make_async_copy(h_hbm.at[a_idx.at[j]], ra[i], sa[i]).wait()
            pltpu.make_async_copy(h_hbm.at[b_idx.at[j]], rb[i], sb[i]).wait()

            @pl.when(j >= 2)
            def _drain():
                pltpu.make_async_copy(
                    oc[i % 2], pred_hbm.at[pl.ds(out_base, CHUNK)],
                    so[i % 2]).wait()

            _dot_chunk_compute(ra[i], rb[i], part, oc[i % 2], lane)
            pltpu.async_copy(oc[i % 2],
                             pred_hbm.at[pl.ds(out_base + j * CHUNK, CHUNK)],
                             so[i % 2])
        return carry

    lax.fori_loop(0, cpt // NBUF, quad, 0)
    for i in range(2):
        pltpu.make_async_copy(oc[i], pred_hbm.at[pl.ds(out_base, CHUNK)],
                              so[i]).wait()


def _dot_phase(h, a2d, b2d):
    mesh = plsc.VectorSubcoreMesh(core_axis_name="c", subcore_axis_name="s",
                                  num_cores=NC, num_subcores=NS)
    fn = pl.kernel(
        _dot_body,
        out_type=jax.ShapeDtypeStruct((NW * OUT_STRIDE,), jnp.float32),
        mesh=mesh,
        scratch_types=(
            [pltpu.VMEM((CPT_MAX, CHUNK), jnp.int32)] * 2
            + [pltpu.VMEM((CHUNK, D // 2), jnp.int32)] * (2 * NBUF)
            + [pltpu.VMEM((16, 16), jnp.float32)]
            + [pltpu.VMEM((CHUNK,), jnp.float32)] * 2
            + [pltpu.SemaphoreType.DMA] * (2 * NBUF + 2)
        ),
        compiler_params=pltpu.CompilerParams(needs_layout_passes=False,
                                             use_tc_tiling_on_sc=False),
    )
    return fn(h, a2d, b2d)


def _pad_idx(idx):
    pad = jnp.full((PAD_ROWS * CHUNK - E,), DUMMY, dtype=idx.dtype)
    return jnp.concatenate([idx, pad]).reshape(PAD_ROWS, CHUNK)


def _unpack_pred(pred_strided):
    per_tile = pred_strided.reshape(NW, OUT_STRIDE)
    sc0 = per_tile[:NS, :CPT0 * CHUNK].reshape(-1)
    sc1 = per_tile[NS:, :CPT1 * CHUNK].reshape(-1)
    return jnp.concatenate([sc0, sc1])[:E]


@jax.jit
def kernel(x, edge_index, edge_label_index, W, W_self):
    x_pad = jnp.concatenate(
        [x, jnp.zeros((N_PAD - N, D), dtype=x.dtype)], axis=0)
    xw, xws = _matmuls(x_pad, W, W_self)

    src2d = _pad_idx(edge_index[0])
    dst2d = _pad_idx(edge_index[1])
    parts = _scatter_phase(xw, xws, src2d, dst2d)
    h = _combine(parts, xws)

    h32 = lax.bitcast_convert_type(h.reshape(N_PAD, D // 2, 2), jnp.int32)

    a2d = _pad_idx(edge_label_index[0])
    b2d = _pad_idx(edge_label_index[1])
    pred_strided = _dot_phase(h32, a2d, b2d)
    return _unpack_pred(pred_strided)


# speedup vs baseline: 1.0008x; 1.0008x over previous
import jax
import jax.numpy as jnp
from jax import lax
from jax.experimental import pallas as pl
from jax.experimental.pallas import tpu as pltpu
from jax.experimental.pallas import tpu_sc as plsc

N = 10000
E = 320000
D = 128

NC = 2
NS = 16
NW = NC * NS

N_PAD = 10112
ROWS_PER_TILE = N_PAD // NS
CHUNK = 128
CPT0 = 80
CPT1 = 80
CPT_MAX = max(CPT0, CPT1)
IDX_ROWS = NS * (CPT0 + CPT1)
E_PAD = IDX_ROWS * CHUNK
PAD_ROWS = NS * CPT0 + (NS - 1) * CPT1 + CPT_MAX
OUT_STRIDE = CPT_MAX * CHUNK
DUMMY = N
DST_BLK = 8


def _core_offsets(c, s):
    cpt = jnp.where(c == 0, CPT0, CPT1)
    base_row = jnp.where(c == 0, s * CPT0, NS * CPT0 + s * CPT1)
    return cpt, base_row


def _mm_body(x_ref, w_ref, ws_ref, xw_ref, xws_ref):
    xb = x_ref[...]
    xw_ref[...] = jnp.dot(xb, w_ref[...], preferred_element_type=jnp.float32)
    xws_ref[...] = jnp.dot(xb, ws_ref[...], preferred_element_type=jnp.float32)


def _matmuls(x_pad, W, W_self):
    blk = N_PAD // 8
    return pl.pallas_call(
        _mm_body,
        grid=(8,),
        in_specs=[
            pl.BlockSpec((blk, D), lambda i: (i, 0)),
            pl.BlockSpec((D, D), lambda i: (0, 0)),
            pl.BlockSpec((D, D), lambda i: (0, 0)),
        ],
        out_specs=[
            pl.BlockSpec((blk, D), lambda i: (i, 0)),
            pl.BlockSpec((blk, D), lambda i: (i, 0)),
        ],
        out_shape=[
            jax.ShapeDtypeStruct((N_PAD, D), jnp.float32),
            jax.ShapeDtypeStruct((N_PAD, D), jnp.float32),
        ],
    )(x_pad, W, W_self)


def _scatter_body(xw_hbm, src_hbm, dst_hbm, parts_hbm,
                  src_idx, dst_idx, rows0, rows1, acc, sem0, sem1, sd0, sd1):
    c = lax.axis_index("c")
    s = lax.axis_index("s")
    cpt, base_row = _core_offsets(c, s)

    zero = jnp.zeros((16,), jnp.float32)
    for r in range(CHUNK):
        for t in range(D // 16):
            rows0[r, pl.ds(16 * t, 16)] = zero
    for k in range(ROWS_PER_TILE // CHUNK):
        pltpu.sync_copy(rows0,
                        acc.at[pl.ds(s * ROWS_PER_TILE + k * CHUNK, CHUNK)])
    _rem = ROWS_PER_TILE % CHUNK
    if _rem:
        pltpu.sync_copy(
            rows0.at[pl.ds(0, _rem)],
            acc.at[pl.ds(s * ROWS_PER_TILE + ROWS_PER_TILE - _rem, _rem)])

    pltpu.sync_copy(src_hbm.at[pl.ds(base_row, CPT_MAX)], src_idx)
    plsc.subcore_barrier()

    bufs = (rows0, rows1)
    sems = (sem0, sem1)
    sd = (sd0, sd1)
    ngroups = cpt // DST_BLK

    pltpu.async_copy(xw_hbm.at[src_idx.at[0]], rows0, sem0)
    pltpu.async_copy(dst_hbm.at[pl.ds(base_row, DST_BLK)], dst_idx.at[0], sd0)

    def gpair(p, carry):
        for par in range(2):
            grp = 2 * p + par
            nk = (par + 1) % 2

            @pl.when(grp + 1 < ngroups)
            def _stage_next():
                pltpu.async_copy(
                    dst_hbm.at[pl.ds(base_row + (grp + 1) * DST_BLK, DST_BLK)],
                    dst_idx.at[nk], sd[nk])

            pltpu.make_async_copy(dst_hbm.at[pl.ds(base_row, DST_BLK)],
                                  dst_idx.at[par], sd[par]).wait()
            for i in range(DST_BLK):
                j = grp * DST_BLK + i

                @pl.when(j + 1 < cpt)
                def _fire():
                    pltpu.async_copy(xw_hbm.at[src_idx.at[j + 1]],
                                     bufs[(i + 1) % 2], sems[(i + 1) % 2])

                pltpu.make_async_copy(xw_hbm.at[src_idx.at[j]], bufs[i % 2],
                                      sems[i % 2]).wait()
                pltpu.sync_copy(bufs[i % 2], acc.at[dst_idx.at[par].at[i]],
                                add=True)
        return carry

    lax.fori_loop(0, ngroups // 2, gpair, 0)
    plsc.subcore_barrier()

    pltpu.sync_copy(acc.at[pl.ds(s * ROWS_PER_TILE, ROWS_PER_TILE)],
                    parts_hbm.at[c].at[pl.ds(s * ROWS_PER_TILE, ROWS_PER_TILE)])


def _scatter_phase(xw, src2d, dst2d):
    mesh = plsc.VectorSubcoreMesh(core_axis_name="c", subcore_axis_name="s",
                                  num_cores=NC, num_subcores=NS)
    fn = pl.kernel(
        _scatter_body,
        out_type=jax.ShapeDtypeStruct((NC, N_PAD, D), jnp.float32),
        mesh=mesh,
        scratch_types=[
            pltpu.VMEM((CPT_MAX, CHUNK), jnp.int32),
            pltpu.VMEM((2, DST_BLK, CHUNK), jnp.int32),
            pltpu.VMEM((CHUNK, D), jnp.float32),
            pltpu.VMEM((CHUNK, D), jnp.float32),
            pltpu.VMEM_SHARED((N_PAD, D), jnp.float32),
            pltpu.SemaphoreType.DMA,
            pltpu.SemaphoreType.DMA,
            pltpu.SemaphoreType.DMA,
            pltpu.SemaphoreType.DMA,
        ],
    )
    return fn(xw, src2d, dst2d)


def _comb_body(p_ref, xws_ref, h_ref):
    h = jnp.maximum(p_ref[0] + p_ref[1] + xws_ref[...], 0.0)
    h_ref[...] = h.astype(jnp.bfloat16)


def _combine(parts, xws):
    blk = N_PAD // 8
    return pl.pallas_call(
        _comb_body,
        grid=(8,),
        in_specs=[
            pl.BlockSpec((NC, blk, D), lambda i: (0, i, 0)),
            pl.BlockSpec((blk, D), lambda i: (i, 0)),
        ],
        out_specs=pl.BlockSpec((blk, D), lambda i: (i, 0)),
        out_shape=jax.ShapeDtypeStruct((N_PAD, D), jnp.bfloat16),
    )(parts, xws)


def _dot_chunk_compute(ra, rb, part, out_c, lane):
    fmt = plsc.PackFormat.INTERLEAVED
    for g in range(CHUNK // 16):
        for r in range(16):
            row = g * 16 + r
            acc = jnp.zeros((16,), jnp.float32)
            for t in range(D // 32):
                va = plsc.bitcast(ra[row, pl.ds(16 * t, 16)], jnp.bfloat16)
                vb = plsc.bitcast(rb[row, pl.ds(16 * t, 16)], jnp.bfloat16)
                a0, a1 = plsc.unpack(va, format=fmt)
                b0, b1 = plsc.unpack(vb, format=fmt)
                acc = acc + a0 * b0 + a1 * b1
            part[r, :] = acc
        dots = jnp.zeros((16,), jnp.float32)
        for t in range(16):
            col = jnp.full((16,), t, jnp.int32)
            dots = dots + plsc.load_gather(part, [lane, col])
        out_c[pl.ds(g * 16, 16)] = dots


NBUF = 4


def _dot_body(h_hbm, a_hbm, b_hbm, pred_hbm,
              a_idx, b_idx, ra0, rb0, ra1, rb1, ra2, rb2, ra3, rb3,
              part, oc0, oc1,
              sa0, sb0, sa1, sb1, sa2, sb2, sa3, sb3, so0, so1):
    c = lax.axis_index("c")
    s = lax.axis_index("s")
    wid = c * NS + s
    cpt, base_row = _core_offsets(c, s)

    pltpu.sync_copy(a_hbm.at[pl.ds(base_row, CPT_MAX)], a_idx)
    pltpu.sync_copy(b_hbm.at[pl.ds(base_row, CPT_MAX)], b_idx)

    lane = lax.iota(jnp.int32, 16)
    ra = (ra0, ra1, ra2, ra3)
    rb = (rb0, rb1, rb2, rb3)
    sa = (sa0, sa1, sa2, sa3)
    sb = (sb0, sb1, sb2, sb3)
    oc = (oc0, oc1)
    so = (so0, so1)
    out_base = wid * OUT_STRIDE

    for k in range(NBUF - 1):
        pltpu.async_copy(h_hbm.at[a_idx.at[k]], ra[k], sa[k])
        pltpu.async_copy(h_hbm.at[b_idx.at[k]], rb[k], sb[k])

    def quad(q, carry):
        for i in range(NBUF):
            j = NBUF * q + i

            @pl.when(j + NBUF - 1 < cpt)
            def _fire():
                k = (i + NBUF - 1) % NBUF
                pltpu.async_copy(h_hbm.at[a_idx.at[j + NBUF - 1]], ra[k],
                                 sa[k])
                pltpu.async_copy(h_hbm.at[b_idx.at[j + NBUF - 1]], rb[k],
                                 sb[k])

            pltpu.make_async_copy(h_hbm.at[a_idx.at[j]], ra[i], sa[i]).wait()
            pltpu.make_async_copy(h_hbm.at[b_idx.at[j]], rb[i], sb[i]).wait()

            @pl.when(j >= 2)
            def _drain():
                pltpu.make_async_copy(
                    oc[i % 2], pred_hbm.at[pl.ds(out_base, CHUNK)],
                    so[i % 2]).wait()

            _dot_chunk_compute(ra[i], rb[i], part, oc[i % 2], lane)
            pltpu.async_copy(oc[i % 2],
                             pred_hbm.at[pl.ds(out_base + j * CHUNK, CHUNK)],
                             so[i % 2])
        return carry

    lax.fori_loop(0, cpt // NBUF, quad, 0)
    for i in range(2):
        pltpu.make_async_copy(oc[i], pred_hbm.at[pl.ds(out_base, CHUNK)],
                              so[i]).wait()


def _dot_phase(h, a2d, b2d):
    mesh = plsc.VectorSubcoreMesh(core_axis_name="c", subcore_axis_name="s",
                                  num_cores=NC, num_subcores=NS)
    fn = pl.kernel(
        _dot_body,
        out_type=jax.ShapeDtypeStruct((NW * OUT_STRIDE,), jnp.float32),
        mesh=mesh,
        scratch_types=(
            [pltpu.VMEM((CPT_MAX, CHUNK), jnp.int32)] * 2
            + [pltpu.VMEM((CHUNK, D // 2), jnp.int32)] * (2 * NBUF)
            + [pltpu.VMEM((16, 16), jnp.float32)]
            + [pltpu.VMEM((CHUNK,), jnp.float32)] * 2
            + [pltpu.SemaphoreType.DMA] * (2 * NBUF + 2)
        ),
        compiler_params=pltpu.CompilerParams(needs_layout_passes=False,
                                             use_tc_tiling_on_sc=False),
    )
    return fn(h, a2d, b2d)


def _pad_idx(idx):
    pad = jnp.full((PAD_ROWS * CHUNK - E,), DUMMY, dtype=idx.dtype)
    return jnp.concatenate([idx, pad]).reshape(PAD_ROWS, CHUNK)


def _unpack_pred(pred_strided):
    per_tile = pred_strided.reshape(NW, OUT_STRIDE)
    sc0 = per_tile[:NS, :CPT0 * CHUNK].reshape(-1)
    sc1 = per_tile[NS:, :CPT1 * CHUNK].reshape(-1)
    return jnp.concatenate([sc0, sc1])[:E]


@jax.jit
def kernel(x, edge_index, edge_label_index, W, W_self):
    x_pad = jnp.concatenate(
        [x, jnp.zeros((N_PAD - N, D), dtype=x.dtype)], axis=0)
    xw, xws = _matmuls(x_pad, W, W_self)

    src2d = _pad_idx(edge_index[0])
    dst2d = _pad_idx(edge_index[1])
    parts = _scatter_phase(xw, src2d, dst2d)
    h = _combine(parts, xws)

    h32 = lax.bitcast_convert_type(h.reshape(N_PAD, D // 2, 2), jnp.int32)

    a2d = _pad_idx(edge_label_index[0])
    b2d = _pad_idx(edge_label_index[1])
    pred_strided = _dot_phase(h32, a2d, b2d)
    return _unpack_pred(pred_strided)


# speedup vs baseline: 1.1571x; 1.1562x over previous
import jax
import jax.numpy as jnp
from jax import lax
from jax.experimental import pallas as pl
from jax.experimental.pallas import tpu as pltpu
from jax.experimental.pallas import tpu_sc as plsc

N = 10000
E = 320000
D = 128

NC = 2
NS = 16
NW = NC * NS

N_PAD = 10112
ROWS_PER_TILE = N_PAD // NS
CHUNK = 128
CPT0 = 112
CPT1 = 48
CPT_MAX = max(CPT0, CPT1)
IDX_ROWS = NS * (CPT0 + CPT1)
E_PAD = IDX_ROWS * CHUNK
PAD_ROWS = NS * CPT0 + (NS - 1) * CPT1 + CPT_MAX
OUT_STRIDE = CPT_MAX * CHUNK
DUMMY = N
DST_BLK = 8


def _core_offsets(c, s):
    cpt = jnp.where(c == 0, CPT0, CPT1)
    base_row = jnp.where(c == 0, s * CPT0, NS * CPT0 + s * CPT1)
    return cpt, base_row


def _mm_body(x_ref, w_ref, ws_ref, xw_ref, xws_ref):
    xb = x_ref[...]
    xw_ref[...] = jnp.dot(xb, w_ref[...], preferred_element_type=jnp.float32)
    xws_ref[...] = jnp.dot(xb, ws_ref[...], preferred_element_type=jnp.float32)


def _matmuls(x_pad, W, W_self):
    blk = N_PAD // 8
    return pl.pallas_call(
        _mm_body,
        grid=(8,),
        in_specs=[
            pl.BlockSpec((blk, D), lambda i: (i, 0)),
            pl.BlockSpec((D, D), lambda i: (0, 0)),
            pl.BlockSpec((D, D), lambda i: (0, 0)),
        ],
        out_specs=[
            pl.BlockSpec((blk, D), lambda i: (i, 0)),
            pl.BlockSpec((blk, D), lambda i: (i, 0)),
        ],
        out_shape=[
            jax.ShapeDtypeStruct((N_PAD, D), jnp.float32),
            jax.ShapeDtypeStruct((N_PAD, D), jnp.float32),
        ],
    )(x_pad, W, W_self)


def _scatter_body(xw_hbm, src_hbm, dst_hbm, parts_hbm,
                  src_idx, dst_idx, rows0, rows1, acc, sem0, sem1, sd0, sd1):
    c = lax.axis_index("c")
    s = lax.axis_index("s")
    cpt, base_row = _core_offsets(c, s)

    zero = jnp.zeros((16,), jnp.float32)
    for r in range(CHUNK):
        for t in range(D // 16):
            rows0[r, pl.ds(16 * t, 16)] = zero
    for k in range(ROWS_PER_TILE // CHUNK):
        pltpu.sync_copy(rows0,
                        acc.at[pl.ds(s * ROWS_PER_TILE + k * CHUNK, CHUNK)])
    _rem = ROWS_PER_TILE % CHUNK
    if _rem:
        pltpu.sync_copy(
            rows0.at[pl.ds(0, _rem)],
            acc.at[pl.ds(s * ROWS_PER_TILE + ROWS_PER_TILE - _rem, _rem)])

    pltpu.sync_copy(src_hbm.at[pl.ds(base_row, CPT_MAX)], src_idx)
    plsc.subcore_barrier()

    bufs = (rows0, rows1)
    sems = (sem0, sem1)
    sd = (sd0, sd1)
    ngroups = cpt // DST_BLK

    pltpu.async_copy(xw_hbm.at[src_idx.at[0]], rows0, sem0)
    pltpu.async_copy(dst_hbm.at[pl.ds(base_row, DST_BLK)], dst_idx.at[0], sd0)

    def gpair(p, carry):
        for par in range(2):
            grp = 2 * p + par
            nk = (par + 1) % 2

            @pl.when(grp + 1 < ngroups)
            def _stage_next():
                pltpu.async_copy(
                    dst_hbm.at[pl.ds(base_row + (grp + 1) * DST_BLK, DST_BLK)],
                    dst_idx.at[nk], sd[nk])

            pltpu.make_async_copy(dst_hbm.at[pl.ds(base_row, DST_BLK)],
                                  dst_idx.at[par], sd[par]).wait()
            for i in range(DST_BLK):
                j = grp * DST_BLK + i

                @pl.when(j + 1 < cpt)
                def _fire():
                    pltpu.async_copy(xw_hbm.at[src_idx.at[j + 1]],
                                     bufs[(i + 1) % 2], sems[(i + 1) % 2])

                pltpu.make_async_copy(xw_hbm.at[src_idx.at[j]], bufs[i % 2],
                                      sems[i % 2]).wait()
                pltpu.sync_copy(bufs[i % 2], acc.at[dst_idx.at[par].at[i]],
                                add=True)
        return carry

    lax.fori_loop(0, ngroups // 2, gpair, 0)
    plsc.subcore_barrier()

    pltpu.sync_copy(acc.at[pl.ds(s * ROWS_PER_TILE, ROWS_PER_TILE)],
                    parts_hbm.at[c].at[pl.ds(s * ROWS_PER_TILE, ROWS_PER_TILE)])


def _scatter_phase(xw, src2d, dst2d):
    mesh = plsc.VectorSubcoreMesh(core_axis_name="c", subcore_axis_name="s",
                                  num_cores=NC, num_subcores=NS)
    fn = pl.kernel(
        _scatter_body,
        out_type=jax.ShapeDtypeStruct((NC, N_PAD, D), jnp.float32),
        mesh=mesh,
        scratch_types=[
            pltpu.VMEM((CPT_MAX, CHUNK), jnp.int32),
            pltpu.VMEM((2, DST_BLK, CHUNK), jnp.int32),
            pltpu.VMEM((CHUNK, D), jnp.float32),
            pltpu.VMEM((CHUNK, D), jnp.float32),
            pltpu.VMEM_SHARED((N_PAD, D), jnp.float32),
            pltpu.SemaphoreType.DMA,
            pltpu.SemaphoreType.DMA,
            pltpu.SemaphoreType.DMA,
            pltpu.SemaphoreType.DMA,
        ],
    )
    return fn(xw, src2d, dst2d)


def _comb_body(p_ref, xws_ref, h_ref):
    h = jnp.maximum(p_ref[0] + p_ref[1] + xws_ref[...], 0.0)
    h_ref[...] = h.astype(jnp.bfloat16)


def _combine(parts, xws):
    blk = N_PAD // 8
    return pl.pallas_call(
        _comb_body,
        grid=(8,),
        in_specs=[
            pl.BlockSpec((NC, blk, D), lambda i: (0, i, 0)),
            pl.BlockSpec((blk, D), lambda i: (i, 0)),
        ],
        out_specs=pl.BlockSpec((blk, D), lambda i: (i, 0)),
        out_shape=jax.ShapeDtypeStruct((N_PAD, D), jnp.bfloat16),
    )(parts, xws)


def _dot_chunk_compute(ra, rb, part, out_c, lane):
    fmt = plsc.PackFormat.INTERLEAVED
    for g in range(CHUNK // 16):
        for r in range(16):
            row = g * 16 + r
            acc = jnp.zeros((16,), jnp.float32)
            for t in range(D // 32):
                va = plsc.bitcast(ra[row, pl.ds(16 * t, 16)], jnp.bfloat16)
                vb = plsc.bitcast(rb[row, pl.ds(16 * t, 16)], jnp.bfloat16)
                a0, a1 = plsc.unpack(va, format=fmt)
                b0, b1 = plsc.unpack(vb, format=fmt)
                acc = acc + a0 * b0 + a1 * b1
            part[r, :] = acc
        dots = jnp.zeros((16,), jnp.float32)
        for t in range(16):
            col = jnp.full((16,), t, jnp.int32)
            dots = dots + plsc.load_gather(part, [lane, col])
        out_c[pl.ds(g * 16, 16)] = dots


NBUF = 4


def _dot_body(h_hbm, a_hbm, b_hbm, pred_hbm,
              a_idx, b_idx, ra0, rb0, ra1, rb1, ra2, rb2, ra3, rb3,
              part, oc0, oc1,
              sa0, sb0, sa1, sb1, sa2, sb2, sa3, sb3, so0, so1):
    c = lax.axis_index("c")
    s = lax.axis_index("s")
    wid = c * NS + s
    cpt, base_row = _core_offsets(c, s)

    pltpu.sync_copy(a_hbm.at[pl.ds(base_row, CPT_MAX)], a_idx)
    pltpu.sync_copy(b_hbm.at[pl.ds(base_row, CPT_MAX)], b_idx)

    lane = lax.iota(jnp.int32, 16)
    ra = (ra0, ra1, ra2, ra3)
    rb = (rb0, rb1, rb2, rb3)
    sa = (sa0, sa1, sa2, sa3)
    sb = (sb0, sb1, sb2, sb3)
    oc = (oc0, oc1)
    so = (so0, so1)
    out_base = wid * OUT_STRIDE

    for k in range(NBUF - 1):
        pltpu.async_copy(h_hbm.at[a_idx.at[k]], ra[k], sa[k])
        pltpu.async_copy(h_hbm.at[b_idx.at[k]], rb[k], sb[k])

    def quad(q, carry):
        for i in range(NBUF):
            j = NBUF * q + i

            @pl.when(j + NBUF - 1 < cpt)
            def _fire():
                k = (i + NBUF - 1) % NBUF
                pltpu.async_copy(h_hbm.at[a_idx.at[j + NBUF - 1]], ra[k],
                                 sa[k])
                pltpu.async_copy(h_hbm.at[b_idx.at[j + NBUF - 1]], rb[k],
                                 sb[k])

            pltpu.make_async_copy(h_hbm.at[a_idx.at[j]], ra[i], sa[i]).wait()
            pltpu.make_async_copy(h_hbm.at[b_idx.at[j]], rb[i], sb[i]).wait()

            @pl.when(j >= 2)
            def _drain():
                pltpu.make_async_copy(
                    oc[i % 2], pred_hbm.at[pl.ds(out_base, CHUNK)],
                    so[i % 2]).wait()

            _dot_chunk_compute(ra[i], rb[i], part, oc[i % 2], lane)
            pltpu.async_copy(oc[i % 2],
                             pred_hbm.at[pl.ds(out_base + j * CHUNK, CHUNK)],
                             so[i % 2])
        return carry

    lax.fori_loop(0, cpt // NBUF, quad, 0)
    for i in range(2):
        pltpu.make_async_copy(oc[i], pred_hbm.at[pl.ds(out_base, CHUNK)],
                              so[i]).wait()


def _dot_phase(h, a2d, b2d):
    mesh = plsc.VectorSubcoreMesh(core_axis_name="c", subcore_axis_name="s",
                                  num_cores=NC, num_subcores=NS)
    fn = pl.kernel(
        _dot_body,
        out_type=jax.ShapeDtypeStruct((NW * OUT_STRIDE,), jnp.float32),
        mesh=mesh,
        scratch_types=(
            [pltpu.VMEM((CPT_MAX, CHUNK), jnp.int32)] * 2
            + [pltpu.VMEM((CHUNK, D // 2), jnp.int32)] * (2 * NBUF)
            + [pltpu.VMEM((16, 16), jnp.float32)]
            + [pltpu.VMEM((CHUNK,), jnp.float32)] * 2
            + [pltpu.SemaphoreType.DMA] * (2 * NBUF + 2)
        ),
        compiler_params=pltpu.CompilerParams(needs_layout_passes=False,
                                             use_tc_tiling_on_sc=False),
    )
    return fn(h, a2d, b2d)


def _pad_idx(idx):
    pad = jnp.full((PAD_ROWS * CHUNK - E,), DUMMY, dtype=idx.dtype)
    return jnp.concatenate([idx, pad]).reshape(PAD_ROWS, CHUNK)


def _unpack_pred(pred_strided):
    per_tile = pred_strided.reshape(NW, OUT_STRIDE)
    sc0 = per_tile[:NS, :CPT0 * CHUNK].reshape(-1)
    sc1 = per_tile[NS:, :CPT1 * CHUNK].reshape(-1)
    return jnp.concatenate([sc0, sc1])[:E]


@jax.jit
def kernel(x, edge_index, edge_label_index, W, W_self):
    x_pad = jnp.concatenate(
        [x, jnp.zeros((N_PAD - N, D), dtype=x.dtype)], axis=0)
    xw, xws = _matmuls(x_pad, W, W_self)

    src2d = _pad_idx(edge_index[0])
    dst2d = _pad_idx(edge_index[1])
    parts = _scatter_phase(xw, src2d, dst2d)
    h = _combine(parts, xws)

    h32 = lax.bitcast_convert_type(h.reshape(N_PAD, D // 2, 2), jnp.int32)

    a2d = _pad_idx(edge_label_index[0])
    b2d = _pad_idx(edge_label_index[1])
    pred_strided = _dot_phase(h32, a2d, b2d)
    return _unpack_pred(pred_strided)
